# baseline (speedup 1.0000x reference)
import functools

import jax
import jax.numpy as jnp
from jax import lax
from jax.experimental import pallas as pl
from jax.experimental.pallas import tpu as pltpu
from jax.experimental.pallas import tpu_sc as plsc

N = 10000
E = 160000
F_IN = 256
F_EDGE = 16
H = 2
C = 256
HC = H * C
EPS = 1e-5
INV_BN = float(1.0 / (1.0 + EPS) ** 0.5)

NP = 10240
NW = 32
EW = E // NW
EWP = EW + 16
NB = (EW + 15) // 16

EC = 8000
NCH = E // EC
WC = 4

_SC_PARAMS = dict(
    compiler_params=pltpu.CompilerParams(needs_layout_passes=False),
)


def _k1a_body(wt_ref, x_ref, w_ref, asrc_ref, adst_ref, o_ref, p_ref):
    xb = x_ref[...]
    o_ref[...] = lax.dot_general(
        wt_ref[...], xb, (((1,), (1,)), ((), ())),
        preferred_element_type=jnp.float32)
    w = w_ref[...]
    us0 = jnp.dot(w[:, :C], asrc_ref[0, :], preferred_element_type=jnp.float32)
    us1 = jnp.dot(w[:, C:], asrc_ref[1, :], preferred_element_type=jnp.float32)
    ud0 = jnp.dot(w[:, :C], adst_ref[0, :], preferred_element_type=jnp.float32)
    ud1 = jnp.dot(w[:, C:], adst_ref[1, :], preferred_element_type=jnp.float32)
    p_ref[0:1, :] = jnp.dot(xb, us0, preferred_element_type=jnp.float32)[None, :]
    p_ref[1:2, :] = jnp.dot(xb, us1, preferred_element_type=jnp.float32)[None, :]
    p_ref[2:3, :] = jnp.dot(xb, ud0, preferred_element_type=jnp.float32)[None, :]
    p_ref[3:4, :] = jnp.dot(xb, ud1, preferred_element_type=jnp.float32)[None, :]
    p_ref[4:8, :] = jnp.zeros((4, p_ref.shape[1]), jnp.float32)


def _xw_and_tables(wt, x, W_gat, att_src, att_dst):
    bn = NP // 8
    f = wt.shape[0]
    return pl.pallas_call(
        _k1a_body,
        grid=(8,),
        in_specs=[pl.BlockSpec((f, F_IN), lambda i: (0, 0)),
                  pl.BlockSpec((bn, F_IN), lambda i: (i, 0)),
                  pl.BlockSpec((F_IN, HC), lambda i: (0, 0)),
                  pl.BlockSpec((H, C), lambda i: (0, 0)),
                  pl.BlockSpec((H, C), lambda i: (0, 0))],
        out_specs=[pl.BlockSpec((f, bn), lambda i: (0, i)),
                   pl.BlockSpec((8, bn), lambda i: (0, i))],
        out_shape=[jax.ShapeDtypeStruct((f, NP), jnp.float32),
                   jax.ShapeDtypeStruct((8, NP), jnp.float32)],
    )(wt, x, W_gat, att_src, att_dst)


def _k1c_body(ea_ref, w_ref, ae_ref, o_ref):
    w = w_ref[...]
    ve0 = jnp.dot(w[:, :C], ae_ref[0, :], preferred_element_type=jnp.float32)
    ve1 = jnp.dot(w[:, C:], ae_ref[1, :], preferred_element_type=jnp.float32)
    ea = ea_ref[...]
    o_ref[0:1, :] = jnp.dot(ea, ve0, preferred_element_type=jnp.float32)[None, :]
    o_ref[1:2, :] = jnp.dot(ea, ve1, preferred_element_type=jnp.float32)[None, :]
    o_ref[2:8, :] = jnp.zeros((6, o_ref.shape[1]), jnp.float32)


def _edge_tables(edge_attr, W_edge, att_edge):
    be = 1280
    return pl.pallas_call(
        _k1c_body,
        grid=(E // be,),
        in_specs=[pl.BlockSpec((be, F_EDGE), lambda i: (i, 0)),
                  pl.BlockSpec((F_EDGE, HC), lambda i: (0, 0)),
                  pl.BlockSpec((H, C), lambda i: (0, 0))],
        out_specs=pl.BlockSpec((8, be), lambda i: (0, i)),
        out_shape=jax.ShapeDtypeStruct((8, E), jnp.float32),
    )(edge_attr, W_edge, att_edge)


def _k2b_body(pd0_ref, pd1_ref, pc_ref, o_ref):
    s0 = jnp.sum(pd0_ref[...], axis=0)
    s1 = jnp.sum(pd1_ref[...], axis=0)
    deg = jnp.sum(pc_ref[...], axis=0) + 1.0
    o_ref[0:1, :] = (s0 + 1e-16)[None, :]
    o_ref[1:2, :] = (s1 + 1e-16)[None, :]
    o_ref[2:3, :] = lax.rsqrt(deg)[None, :]
    o_ref[3:4, :] = (1.0 / deg)[None, :]
    o_ref[4:8, :] = jnp.zeros((4, o_ref.shape[1]), jnp.float32)


def _reduce_tables(pd0, pd1, pcnt):
    bn = NP // 8
    return pl.pallas_call(
        _k2b_body,
        grid=(8,),
        in_specs=[pl.BlockSpec((NW, bn), lambda i: (0, i)),
                  pl.BlockSpec((NW, bn), lambda i: (0, i)),
                  pl.BlockSpec((NW, bn), lambda i: (0, i))],
        out_specs=pl.BlockSpec((8, bn), lambda i: (0, i)),
        out_shape=jax.ShapeDtypeStruct((8, NP), jnp.float32),
    )(pd0, pd1, pcnt)


def _k4_body(agg_ref, b_ref, g_ref, be_ref, wt_ref, o_ref):
    hb = agg_ref[...] + b_ref[...]
    hb = hb * (INV_BN * g_ref[...]) + be_ref[...]
    hb = jnp.where(hb > 0, hb, jnp.exp(hb) - 1.0)
    o_ref[...] = lax.dot_general(
        wt_ref[...], hb, (((1,), (0,)), ((), ())),
        preferred_element_type=jnp.float32)


def _fuse_matmul_t(aggt, bcol, gcol, becol, wt):
    bn = NP // 8
    f = aggt.shape[0]
    fo = wt.shape[0]
    return pl.pallas_call(
        _k4_body,
        grid=(8,),
        in_specs=[pl.BlockSpec((f, bn), lambda i: (0, i)),
                  pl.BlockSpec((f, 1), lambda i: (0, 0)),
                  pl.BlockSpec((f, 1), lambda i: (0, 0)),
                  pl.BlockSpec((f, 1), lambda i: (0, 0)),
                  pl.BlockSpec((fo, f), lambda i: (0, 0))],
        out_specs=pl.BlockSpec((fo, bn), lambda i: (0, i)),
        out_shape=jax.ShapeDtypeStruct((fo, NP), jnp.float32),
    )(aggt, bcol, gcol, becol, wt)


def _k8_body(agg_ref, x_ref, b_ref, o_ref, m_ref):
    net = agg_ref[...] + b_ref[...]
    r = lax.broadcasted_iota(jnp.int32, (C, C), 0)
    c = lax.broadcasted_iota(jnp.int32, (C, C), 1)
    eye = (r == c).astype(jnp.float32)
    ne = lax.dot_general(net, eye, (((0,), (0,)), ((), ())),
                         preferred_element_type=jnp.float32)
    ne = ne + x_ref[...]
    o_ref[...] = ne
    i = pl.program_id(0)
    bn = ne.shape[0]

    @pl.when(i == 0)
    def _():
        m_ref[...] = jnp.zeros((1, C), jnp.float32)

    rows = lax.broadcasted_iota(jnp.int32, (bn, C), 0) + i * bn
    m_ref[...] += jnp.sum(jnp.where(rows < N, ne, 0.0), axis=0,
                          keepdims=True)

    @pl.when(i == pl.num_programs(0) - 1)
    def _():
        m_ref[...] *= (1.0 / N)


def _final(agg2t, x, b2col):
    bn = NP // 8
    return pl.pallas_call(
        _k8_body,
        grid=(8,),
        in_specs=[pl.BlockSpec((C, bn), lambda i: (0, i)),
                  pl.BlockSpec((bn, C), lambda i: (i, 0)),
                  pl.BlockSpec((C, 1), lambda i: (0, 0))],
        out_specs=[pl.BlockSpec((bn, C), lambda i: (i, 0)),
                   pl.BlockSpec((1, C), lambda i: (0, 0))],
        out_shape=[jax.ShapeDtypeStruct((N, C), jnp.float32),
                   jax.ShapeDtypeStruct((1, C), jnp.float32)],
    )(agg2t, x, b2col)


def _sc_mesh():
    return plsc.VectorSubcoreMesh(core_axis_name="c", subcore_axis_name="s")


def _worker_prologue():
    cid = lax.axis_index("c")
    sid = lax.axis_index("s")
    wid = sid * 2 + cid
    return cid, sid, wid, wid * EW


def _exp_denom_pass(src, dst, aed, ptab):
    kfn = functools.partial(
        pl.kernel,
        out_type=(jax.ShapeDtypeStruct((E,), jnp.float32),
                  jax.ShapeDtypeStruct((E,), jnp.float32),
                  jax.ShapeDtypeStruct((NW * NP,), jnp.float32),
                  jax.ShapeDtypeStruct((NW * NP,), jnp.float32),
                  jax.ShapeDtypeStruct((NW * NP,), jnp.float32)),
        mesh=_sc_mesh(),
        scratch_types=[
            pltpu.VMEM((NP,), jnp.float32),
            pltpu.VMEM((NP,), jnp.float32),
            pltpu.VMEM((NP,), jnp.float32),
            pltpu.VMEM((NP,), jnp.float32),
            pltpu.VMEM((EWP,), jnp.int32),
            pltpu.VMEM((EWP,), jnp.int32),
            pltpu.VMEM((EWP,), jnp.float32),
            pltpu.VMEM((EWP,), jnp.float32),
            pltpu.VMEM((EWP,), jnp.float32),
            pltpu.VMEM((EWP,), jnp.float32),
            pltpu.VMEM((NP,), jnp.float32),
            pltpu.VMEM((NP,), jnp.float32),
            pltpu.VMEM((NP,), jnp.float32),
        ],
        **_SC_PARAMS,
    )

    @kfn
    def body(src_h, dst_h, aed_h, ptab_h, exp0_h, exp1_h, pd0_h, pd1_h, pcn_h,
             as0, as1, ad0, ad1, srcv, dstv, ae0, ae1, ex0, ex1, d0, d1, cn):
        _, _, wid, base = _worker_prologue()
        pltpu.sync_copy(ptab_h.at[pl.ds(0 * NP, NP)], as0)
        pltpu.sync_copy(ptab_h.at[pl.ds(1 * NP, NP)], as1)
        pltpu.sync_copy(ptab_h.at[pl.ds(2 * NP, NP)], ad0)
        pltpu.sync_copy(ptab_h.at[pl.ds(3 * NP, NP)], ad1)
        pltpu.sync_copy(src_h.at[pl.ds(base, EW)], srcv.at[pl.ds(0, EW)])
        pltpu.sync_copy(dst_h.at[pl.ds(base, EW)], dstv.at[pl.ds(0, EW)])
        pltpu.sync_copy(aed_h.at[pl.ds(0 * E + base, EW)], ae0.at[pl.ds(0, EW)])
        pltpu.sync_copy(aed_h.at[pl.ds(1 * E + base, EW)], ae1.at[pl.ds(0, EW)])

        zf = jnp.zeros((16,), jnp.float32)

        @plsc.parallel_loop(0, NP, 16, unroll=4)
        def _(off):
            d0[pl.ds(off, 16)] = zf
            d1[pl.ds(off, 16)] = zf
            cn[pl.ds(off, 16)] = zf

        iota = lax.iota(jnp.int32, 16)
        ones = jnp.ones((16,), jnp.float32)

        def ebody(b, _):
            off = b * 16
            m = (off + iota) < EW
            sv = jnp.where(m, srcv[pl.ds(off, 16)], 0)
            dv = jnp.where(m, dstv[pl.ds(off, 16)], 0)
            a0 = (plsc.load_gather(as0, [sv]) + plsc.load_gather(ad0, [dv])
                  + ae0[pl.ds(off, 16)])
            a1 = (plsc.load_gather(as1, [sv]) + plsc.load_gather(ad1, [dv])
                  + ae1[pl.ds(off, 16)])
            a0 = jnp.where(a0 >= 0.0, a0, a0 * 0.2)
            a1 = jnp.where(a1 >= 0.0, a1, a1 * 0.2)
            e0 = jnp.exp(a0)
            e1 = jnp.exp(a1)
            ex0[pl.ds(off, 16)] = e0
            ex1[pl.ds(off, 16)] = e1
            plsc.addupdate_scatter(d0, [dv], e0, mask=m)
            plsc.addupdate_scatter(d1, [dv], e1, mask=m)
            plsc.addupdate_scatter(cn, [dv], ones, mask=m)
            return 0

        lax.fori_loop(0, NB, ebody, 0)
        pltpu.sync_copy(ex0.at[pl.ds(0, EW)], exp0_h.at[pl.ds(base, EW)])
        pltpu.sync_copy(ex1.at[pl.ds(0, EW)], exp1_h.at[pl.ds(base, EW)])
        pltpu.sync_copy(d0, pd0_h.at[pl.ds(wid * NP, NP)])
        pltpu.sync_copy(d1, pd1_h.at[pl.ds(wid * NP, NP)])
        pltpu.sync_copy(cn, pcn_h.at[pl.ds(wid * NP, NP)])

    return body(src, dst, aed, ptab)


def _weight_pass(src, dst, exp0, exp1, qtab):
    kfn = functools.partial(
        pl.kernel,
        out_type=(jax.ShapeDtypeStruct((2 * E,), jnp.float32),
                  jax.ShapeDtypeStruct((E,), jnp.float32),
                  jax.ShapeDtypeStruct((E,), jnp.int32)),
        mesh=_sc_mesh(),
        scratch_types=[
            pltpu.VMEM((NP,), jnp.float32),
            pltpu.VMEM((NP,), jnp.float32),
            pltpu.VMEM((NP,), jnp.float32),
            pltpu.VMEM((EWP,), jnp.int32),
            pltpu.VMEM((EWP,), jnp.int32),
            pltpu.VMEM((EWP,), jnp.float32),
            pltpu.VMEM((EWP,), jnp.float32),
            pltpu.VMEM((EWP,), jnp.float32),
            pltpu.VMEM((EWP,), jnp.float32),
            pltpu.VMEM((EWP,), jnp.float32),
            pltpu.VMEM((EWP,), jnp.int32),
        ],
        **_SC_PARAMS,
    )

    @kfn
    def body(src_h, dst_h, exp0_h, exp1_h, qtab_h, w01_h, wn_h, sd_h,
             q0, q1, disv, srcv, dstv, ex0, ex1, w0, w1, wn, sdv):
        _, _, wid, base = _worker_prologue()
        pltpu.sync_copy(qtab_h.at[pl.ds(0 * NP, NP)], q0)
        pltpu.sync_copy(qtab_h.at[pl.ds(1 * NP, NP)], q1)
        pltpu.sync_copy(qtab_h.at[pl.ds(2 * NP, NP)], disv)
        pltpu.sync_copy(src_h.at[pl.ds(base, EW)], srcv.at[pl.ds(0, EW)])
        pltpu.sync_copy(dst_h.at[pl.ds(base, EW)], dstv.at[pl.ds(0, EW)])
        pltpu.sync_copy(exp0_h.at[pl.ds(base, EW)], ex0.at[pl.ds(0, EW)])
        pltpu.sync_copy(exp1_h.at[pl.ds(base, EW)], ex1.at[pl.ds(0, EW)])

        iota = lax.iota(jnp.int32, 16)

        def ebody(b, _):
            off = b * 16
            m = (off + iota) < EW
            sv = jnp.where(m, srcv[pl.ds(off, 16)], 0)
            dv = jnp.where(m, dstv[pl.ds(off, 16)], 0)
            w0[pl.ds(off, 16)] = ex0[pl.ds(off, 16)] / plsc.load_gather(q0, [dv])
            w1[pl.ds(off, 16)] = ex1[pl.ds(off, 16)] / plsc.load_gather(q1, [dv])
            wn[pl.ds(off, 16)] = (plsc.load_gather(disv, [sv])
                                  * plsc.load_gather(disv, [dv]))
            sdv[pl.ds(off, 16)] = dv * 65536 + sv
            return 0

        lax.fori_loop(0, NB, ebody, 0)
        pltpu.sync_copy(w0.at[pl.ds(0, EW)], w01_h.at[pl.ds(base, EW)])
        pltpu.sync_copy(w1.at[pl.ds(0, EW)], w01_h.at[pl.ds(E + base, EW)])
        pltpu.sync_copy(wn.at[pl.ds(0, EW)], wn_h.at[pl.ds(base, EW)])
        pltpu.sync_copy(sdv.at[pl.ds(0, EW)], sd_h.at[pl.ds(base, EW)])

    return body(src, dst, exp0, exp1, qtab)


def _gat_aggregate(sd, w01, xwt):
    kfn = functools.partial(
        pl.kernel,
        out_type=jax.ShapeDtypeStruct((HC * NP,), jnp.float32),
        mesh=_sc_mesh(),
        scratch_types=(
            [pltpu.VMEM((NP,), jnp.float32) for _ in range(2 * WC)]
            + [pltpu.VMEM((EC,), jnp.int32), pltpu.VMEM((EC,), jnp.float32),
               pltpu.VMEM((EC,), jnp.int32), pltpu.VMEM((EC,), jnp.float32),
               pltpu.SemaphoreType.DMA, pltpu.SemaphoreType.DMA]),
        **_SC_PARAMS,
    )

    @kfn
    def body(sd_h, w01_h, xwt_h, agg_h,
             xc0, xc1, xc2, xc3, ac0, ac1, ac2, ac3,
             sb0, wb0, sb1, wb1, sem0, sem1):
        _, _, wid, _ = _worker_prologue()
        xc = [xc0, xc1, xc2, xc3]
        ac = [ac0, ac1, ac2, ac3]
        bufs = [(sb0, wb0, sem0), (sb1, wb1, sem1)]
        zf = jnp.zeros((16,), jnp.float32)
        woff = jnp.where(wid < 16, 0, E)

        def issue(par, eb):
            sb, wb, sem = bufs[par]
            pltpu.async_copy(sd_h.at[pl.ds(eb, EC)], sb, sem)
            pltpu.async_copy(w01_h.at[pl.ds(woff + eb, EC)], wb, sem)

        def drain(par):
            sb, wb, sem = bufs[par]
            pltpu.make_async_copy(sd_h.at[pl.ds(0, EC)], sb, sem).wait()
            pltpu.make_async_copy(w01_h.at[pl.ds(0, EC)], wb, sem).wait()

        def process(par):
            sb, wb, _ = bufs[par]

            @plsc.parallel_loop(0, EC, 16, unroll=16)
            def _(off):
                sd_v = sb[pl.ds(off, 16)]
                sv = jnp.bitwise_and(sd_v, 65535)
                dv = lax.shift_right_logical(sd_v, 16)
                wv = wb[pl.ds(off, 16)]
                for k in range(WC):
                    val = plsc.load_gather(xc[k], [sv]) * wv
                    plsc.addupdate_scatter(ac[k], [dv], val)

        for p in range(16 // WC):
            col0 = wid * 16 + p * WC
            for k in range(WC):
                pltpu.sync_copy(xwt_h.at[pl.ds((col0 + k) * NP, NP)], xc[k])

            @plsc.parallel_loop(0, NP, 16, unroll=4)
            def _(off):
                for k in range(WC):
                    ac[k][pl.ds(off, 16)] = zf

            issue(0, 0)

            def cbody(ci, _):
                nxt = ci + 1

                @pl.when(ci % 2 == 0)
                def _():
                    @pl.when(nxt < NCH)
                    def _():
                        issue(1, nxt * EC)
                    drain(0)
                    process(0)

                @pl.when(ci % 2 == 1)
                def _():
                    @pl.when(nxt < NCH)
                    def _():
                        issue(0, nxt * EC)
                    drain(1)
                    process(1)

                return 0

            lax.fori_loop(0, NCH, cbody, 0)
            for k in range(WC):
                pltpu.sync_copy(ac[k], agg_h.at[pl.ds((col0 + k) * NP, NP)])

    return body(sd, w01, xwt)


def _gcn_aggregate(sd, wn, qtab, hwt):
    kfn = functools.partial(
        pl.kernel,
        out_type=jax.ShapeDtypeStruct((C * NP,), jnp.float32),
        mesh=_sc_mesh(),
        scratch_types=(
            [pltpu.VMEM((NP,), jnp.float32) for _ in range(2 * WC)]
            + [pltpu.VMEM((NP,), jnp.float32),
               pltpu.VMEM((EC,), jnp.int32), pltpu.VMEM((EC,), jnp.float32),
               pltpu.VMEM((EC,), jnp.int32), pltpu.VMEM((EC,), jnp.float32),
               pltpu.SemaphoreType.DMA, pltpu.SemaphoreType.DMA]),
        **_SC_PARAMS,
    )

    @kfn
    def body(sd_h, wn_h, qtab_h, hwt_h, agg_h,
             xc0, xc1, xc2, xc3, ac0, ac1, ac2, ac3, swv,
             sb0, wb0, sb1, wb1, sem0, sem1):
        _, _, wid, _ = _worker_prologue()
        xc = [xc0, xc1, xc2, xc3]
        ac = [ac0, ac1, ac2, ac3]
        bufs = [(sb0, wb0, sem0), (sb1, wb1, sem1)]
        pltpu.sync_copy(qtab_h.at[pl.ds(3 * NP, NP)], swv)
        zf = jnp.zeros((16,), jnp.float32)

        def issue(par, eb):
            sb, wb, sem = bufs[par]
            pltpu.async_copy(sd_h.at[pl.ds(eb, EC)], sb, sem)
            pltpu.async_copy(wn_h.at[pl.ds(eb, EC)], wb, sem)

        def drain(par):
            sb, wb, sem = bufs[par]
            pltpu.make_async_copy(sd_h.at[pl.ds(0, EC)], sb, sem).wait()
            pltpu.make_async_copy(wn_h.at[pl.ds(0, EC)], wb, sem).wait()

        def process(par):
            sb, wb, _ = bufs[par]

            @plsc.parallel_loop(0, EC, 16, unroll=16)
            def _(off):
                sd_v = sb[pl.ds(off, 16)]
                sv = jnp.bitwise_and(sd_v, 65535)
                dv = lax.shift_right_logical(sd_v, 16)
                wv = wb[pl.ds(off, 16)]
                for k in range(WC):
                    val = plsc.load_gather(xc[k], [sv]) * wv
                    plsc.addupdate_scatter(ac[k], [dv], val)

        for p in range(8 // WC):
            col0 = wid * 8 + p * WC
            for k in range(WC):
                pltpu.sync_copy(hwt_h.at[pl.ds((col0 + k) * NP, NP)], xc[k])

            @plsc.parallel_loop(0, NP, 16, unroll=4)
            def _(off):
                for k in range(WC):
                    ac[k][pl.ds(off, 16)] = zf

            issue(0, 0)

            def cbody(ci, _):
                nxt = ci + 1

                @pl.when(ci % 2 == 0)
                def _():
                    @pl.when(nxt < NCH)
                    def _():
                        issue(1, nxt * EC)
                    drain(0)
                    process(0)

                @pl.when(ci % 2 == 1)
                def _():
                    @pl.when(nxt < NCH)
                    def _():
                        issue(0, nxt * EC)
                    drain(1)
                    process(1)

                return 0

            lax.fori_loop(0, NCH, cbody, 0)

            @plsc.parallel_loop(0, NP, 16, unroll=4)
            def _(off):
                swl = swv[pl.ds(off, 16)]
                for k in range(WC):
                    ac[k][pl.ds(off, 16)] = (ac[k][pl.ds(off, 16)]
                                             + swl * xc[k][pl.ds(off, 16)])

            for k in range(WC):
                pltpu.sync_copy(ac[k], agg_h.at[pl.ds((col0 + k) * NP, NP)])

    return body(sd, wn, qtab, hwt)


def kernel(x, edge_index, edge_attr, W_gat, b_gat, att_src, att_dst, att_edge,
           W_edge, gamma1, beta1, gamma2, beta2, W1, b1, W2, b2):
    src = edge_index[0]
    dst = edge_index[1]
    xwt, ptab = _xw_and_tables(W_gat.T, x, W_gat, att_src, att_dst)
    xwt = xwt.reshape(HC * NP)
    ptab = ptab.reshape(8 * NP)
    aed = _edge_tables(edge_attr, W_edge, att_edge).reshape(8 * E)
    exp0, exp1, pd0, pd1, pcnt = _exp_denom_pass(src, dst, aed, ptab)
    qtab = _reduce_tables(pd0.reshape(NW, NP), pd1.reshape(NW, NP),
                          pcnt.reshape(NW, NP)).reshape(8 * NP)
    w01, wn, sd = _weight_pass(src, dst, exp0, exp1, qtab)
    aggt = _gat_aggregate(sd, w01, xwt).reshape(HC, NP)
    h1wt = _fuse_matmul_t(aggt, b_gat[:, None], gamma1[:, None],
                          beta1[:, None], W1.T).reshape(C * NP)
    agg1t = _gcn_aggregate(sd, wn, qtab, h1wt).reshape(C, NP)
    h2wt = _fuse_matmul_t(agg1t, b1[:, None], gamma2[:, None],
                          beta2[:, None], W2.T).reshape(C * NP)
    agg2t = _gcn_aggregate(sd, wn, qtab, h2wt).reshape(C, NP)
    node_emb, graph_emb = _final(agg2t, x, b2[:, None])
    return (node_emb, graph_emb)


# speedup vs baseline: 1.0161x; 1.0161x over previous
import functools

import jax
import jax.numpy as jnp
from jax import lax
from jax.experimental import pallas as pl
from jax.experimental.pallas import tpu as pltpu
from jax.experimental.pallas import tpu_sc as plsc

N = 10000
E = 160000
F_IN = 256
F_EDGE = 16
H = 2
C = 256
HC = H * C
EPS = 1e-5
INV_BN = float(1.0 / (1.0 + EPS) ** 0.5)

NP = 10240
NW = 32
EW = E // NW
EWP = EW + 16
NB = (EW + 15) // 16

EC = 8000
NCH = E // EC
WC = 4

_SC_PARAMS = dict(
    compiler_params=pltpu.CompilerParams(needs_layout_passes=False),
)


def _k1a_body(wt_ref, x_ref, w_ref, asrc_ref, adst_ref, o_ref, p_ref):
    xb = x_ref[...]
    o_ref[...] = lax.dot_general(
        wt_ref[...], xb, (((1,), (1,)), ((), ())),
        preferred_element_type=jnp.float32)
    w = w_ref[...]
    us0 = jnp.dot(w[:, :C], asrc_ref[0, :], preferred_element_type=jnp.float32)
    us1 = jnp.dot(w[:, C:], asrc_ref[1, :], preferred_element_type=jnp.float32)
    ud0 = jnp.dot(w[:, :C], adst_ref[0, :], preferred_element_type=jnp.float32)
    ud1 = jnp.dot(w[:, C:], adst_ref[1, :], preferred_element_type=jnp.float32)
    p_ref[0:1, :] = jnp.dot(xb, us0, preferred_element_type=jnp.float32)[None, :]
    p_ref[1:2, :] = jnp.dot(xb, us1, preferred_element_type=jnp.float32)[None, :]
    p_ref[2:3, :] = jnp.dot(xb, ud0, preferred_element_type=jnp.float32)[None, :]
    p_ref[3:4, :] = jnp.dot(xb, ud1, preferred_element_type=jnp.float32)[None, :]
    p_ref[4:8, :] = jnp.zeros((4, p_ref.shape[1]), jnp.float32)


def _xw_and_tables(wt, x, W_gat, att_src, att_dst):
    bn = NP // 8
    f = wt.shape[0]
    return pl.pallas_call(
        _k1a_body,
        grid=(8,),
        in_specs=[pl.BlockSpec((f, F_IN), lambda i: (0, 0)),
                  pl.BlockSpec((bn, F_IN), lambda i: (i, 0)),
                  pl.BlockSpec((F_IN, HC), lambda i: (0, 0)),
                  pl.BlockSpec((H, C), lambda i: (0, 0)),
                  pl.BlockSpec((H, C), lambda i: (0, 0))],
        out_specs=[pl.BlockSpec((f, bn), lambda i: (0, i)),
                   pl.BlockSpec((8, bn), lambda i: (0, i))],
        out_shape=[jax.ShapeDtypeStruct((f, NP), jnp.float32),
                   jax.ShapeDtypeStruct((8, NP), jnp.float32)],
    )(wt, x, W_gat, att_src, att_dst)


def _k1c_body(ea_ref, w_ref, ae_ref, o_ref):
    w = w_ref[...]
    ve0 = jnp.dot(w[:, :C], ae_ref[0, :], preferred_element_type=jnp.float32)
    ve1 = jnp.dot(w[:, C:], ae_ref[1, :], preferred_element_type=jnp.float32)
    ea = ea_ref[...]
    o_ref[0:1, :] = jnp.dot(ea, ve0, preferred_element_type=jnp.float32)[None, :]
    o_ref[1:2, :] = jnp.dot(ea, ve1, preferred_element_type=jnp.float32)[None, :]
    o_ref[2:8, :] = jnp.zeros((6, o_ref.shape[1]), jnp.float32)


def _edge_tables(edge_attr, W_edge, att_edge):
    be = 1280
    return pl.pallas_call(
        _k1c_body,
        grid=(E // be,),
        in_specs=[pl.BlockSpec((be, F_EDGE), lambda i: (i, 0)),
                  pl.BlockSpec((F_EDGE, HC), lambda i: (0, 0)),
                  pl.BlockSpec((H, C), lambda i: (0, 0))],
        out_specs=pl.BlockSpec((8, be), lambda i: (0, i)),
        out_shape=jax.ShapeDtypeStruct((8, E), jnp.float32),
    )(edge_attr, W_edge, att_edge)


def _k2b_body(pd0_ref, pd1_ref, pc_ref, o_ref):
    s0 = jnp.sum(pd0_ref[...], axis=0)
    s1 = jnp.sum(pd1_ref[...], axis=0)
    deg = jnp.sum(pc_ref[...], axis=0) + 1.0
    o_ref[0:1, :] = (s0 + 1e-16)[None, :]
    o_ref[1:2, :] = (s1 + 1e-16)[None, :]
    o_ref[2:3, :] = lax.rsqrt(deg)[None, :]
    o_ref[3:4, :] = (1.0 / deg)[None, :]
    o_ref[4:8, :] = jnp.zeros((4, o_ref.shape[1]), jnp.float32)


def _reduce_tables(pd0, pd1, pcnt):
    bn = NP // 8
    return pl.pallas_call(
        _k2b_body,
        grid=(8,),
        in_specs=[pl.BlockSpec((NW, bn), lambda i: (0, i)),
                  pl.BlockSpec((NW, bn), lambda i: (0, i)),
                  pl.BlockSpec((NW, bn), lambda i: (0, i))],
        out_specs=pl.BlockSpec((8, bn), lambda i: (0, i)),
        out_shape=jax.ShapeDtypeStruct((8, NP), jnp.float32),
    )(pd0, pd1, pcnt)


def _k4_body(agg_ref, b_ref, g_ref, be_ref, wt_ref, o_ref):
    hb = agg_ref[...] + b_ref[...]
    hb = hb * (INV_BN * g_ref[...]) + be_ref[...]
    hb = jnp.where(hb > 0, hb, jnp.exp(hb) - 1.0)
    o_ref[...] = lax.dot_general(
        wt_ref[...], hb, (((1,), (0,)), ((), ())),
        preferred_element_type=jnp.float32)


def _fuse_matmul_t(aggt, bcol, gcol, becol, wt):
    bn = NP // 8
    f = aggt.shape[0]
    fo = wt.shape[0]
    return pl.pallas_call(
        _k4_body,
        grid=(8,),
        in_specs=[pl.BlockSpec((f, bn), lambda i: (0, i)),
                  pl.BlockSpec((f, 1), lambda i: (0, 0)),
                  pl.BlockSpec((f, 1), lambda i: (0, 0)),
                  pl.BlockSpec((f, 1), lambda i: (0, 0)),
                  pl.BlockSpec((fo, f), lambda i: (0, 0))],
        out_specs=pl.BlockSpec((fo, bn), lambda i: (0, i)),
        out_shape=jax.ShapeDtypeStruct((fo, NP), jnp.float32),
    )(aggt, bcol, gcol, becol, wt)


def _k8_body(agg_ref, x_ref, b_ref, o_ref, m_ref):
    net = agg_ref[...] + b_ref[...]
    r = lax.broadcasted_iota(jnp.int32, (C, C), 0)
    c = lax.broadcasted_iota(jnp.int32, (C, C), 1)
    eye = (r == c).astype(jnp.float32)
    ne = lax.dot_general(net, eye, (((0,), (0,)), ((), ())),
                         preferred_element_type=jnp.float32)
    ne = ne + x_ref[...]
    o_ref[...] = ne
    i = pl.program_id(0)
    bn = ne.shape[0]

    @pl.when(i == 0)
    def _():
        m_ref[...] = jnp.zeros((1, C), jnp.float32)

    rows = lax.broadcasted_iota(jnp.int32, (bn, C), 0) + i * bn
    m_ref[...] += jnp.sum(jnp.where(rows < N, ne, 0.0), axis=0,
                          keepdims=True)

    @pl.when(i == pl.num_programs(0) - 1)
    def _():
        m_ref[...] *= (1.0 / N)


def _final(agg2t, x, b2col):
    bn = NP // 8
    return pl.pallas_call(
        _k8_body,
        grid=(8,),
        in_specs=[pl.BlockSpec((C, bn), lambda i: (0, i)),
                  pl.BlockSpec((bn, C), lambda i: (i, 0)),
                  pl.BlockSpec((C, 1), lambda i: (0, 0))],
        out_specs=[pl.BlockSpec((bn, C), lambda i: (i, 0)),
                   pl.BlockSpec((1, C), lambda i: (0, 0))],
        out_shape=[jax.ShapeDtypeStruct((N, C), jnp.float32),
                   jax.ShapeDtypeStruct((1, C), jnp.float32)],
    )(agg2t, x, b2col)


def _sc_mesh():
    return plsc.VectorSubcoreMesh(core_axis_name="c", subcore_axis_name="s")


def _worker_prologue():
    cid = lax.axis_index("c")
    sid = lax.axis_index("s")
    wid = sid * 2 + cid
    return cid, sid, wid, wid * EW


def _exp_denom_pass(src, dst, aed, ptab):
    kfn = functools.partial(
        pl.kernel,
        out_type=(jax.ShapeDtypeStruct((E,), jnp.float32),
                  jax.ShapeDtypeStruct((E,), jnp.float32),
                  jax.ShapeDtypeStruct((NW * NP,), jnp.float32),
                  jax.ShapeDtypeStruct((NW * NP,), jnp.float32),
                  jax.ShapeDtypeStruct((NW * NP,), jnp.float32)),
        mesh=_sc_mesh(),
        scratch_types=[
            pltpu.VMEM((NP,), jnp.float32),
            pltpu.VMEM((NP,), jnp.float32),
            pltpu.VMEM((NP,), jnp.float32),
            pltpu.VMEM((NP,), jnp.float32),
            pltpu.VMEM((EWP,), jnp.int32),
            pltpu.VMEM((EWP,), jnp.int32),
            pltpu.VMEM((EWP,), jnp.float32),
            pltpu.VMEM((EWP,), jnp.float32),
            pltpu.VMEM((EWP,), jnp.float32),
            pltpu.VMEM((EWP,), jnp.float32),
            pltpu.VMEM((NP,), jnp.float32),
            pltpu.VMEM((NP,), jnp.float32),
            pltpu.VMEM((NP,), jnp.float32),
        ],
        **_SC_PARAMS,
    )

    @kfn
    def body(src_h, dst_h, aed_h, ptab_h, exp0_h, exp1_h, pd0_h, pd1_h, pcn_h,
             as0, as1, ad0, ad1, srcv, dstv, ae0, ae1, ex0, ex1, d0, d1, cn):
        _, _, wid, base = _worker_prologue()
        pltpu.sync_copy(ptab_h.at[pl.ds(0 * NP, NP)], as0)
        pltpu.sync_copy(ptab_h.at[pl.ds(1 * NP, NP)], as1)
        pltpu.sync_copy(ptab_h.at[pl.ds(2 * NP, NP)], ad0)
        pltpu.sync_copy(ptab_h.at[pl.ds(3 * NP, NP)], ad1)
        pltpu.sync_copy(src_h.at[pl.ds(base, EW)], srcv.at[pl.ds(0, EW)])
        pltpu.sync_copy(dst_h.at[pl.ds(base, EW)], dstv.at[pl.ds(0, EW)])
        pltpu.sync_copy(aed_h.at[pl.ds(0 * E + base, EW)], ae0.at[pl.ds(0, EW)])
        pltpu.sync_copy(aed_h.at[pl.ds(1 * E + base, EW)], ae1.at[pl.ds(0, EW)])

        zf = jnp.zeros((16,), jnp.float32)

        @plsc.parallel_loop(0, NP, 16, unroll=4)
        def _(off):
            d0[pl.ds(off, 16)] = zf
            d1[pl.ds(off, 16)] = zf
            cn[pl.ds(off, 16)] = zf

        iota = lax.iota(jnp.int32, 16)
        ones = jnp.ones((16,), jnp.float32)

        def ebody(b, _):
            off = b * 16
            m = (off + iota) < EW
            sv = jnp.where(m, srcv[pl.ds(off, 16)], 0)
            dv = jnp.where(m, dstv[pl.ds(off, 16)], 0)
            a0 = (plsc.load_gather(as0, [sv]) + plsc.load_gather(ad0, [dv])
                  + ae0[pl.ds(off, 16)])
            a1 = (plsc.load_gather(as1, [sv]) + plsc.load_gather(ad1, [dv])
                  + ae1[pl.ds(off, 16)])
            a0 = jnp.where(a0 >= 0.0, a0, a0 * 0.2)
            a1 = jnp.where(a1 >= 0.0, a1, a1 * 0.2)
            e0 = jnp.exp(a0)
            e1 = jnp.exp(a1)
            ex0[pl.ds(off, 16)] = e0
            ex1[pl.ds(off, 16)] = e1
            plsc.addupdate_scatter(d0, [dv], e0, mask=m)
            plsc.addupdate_scatter(d1, [dv], e1, mask=m)
            plsc.addupdate_scatter(cn, [dv], ones, mask=m)
            return 0

        lax.fori_loop(0, NB, ebody, 0)
        pltpu.sync_copy(ex0.at[pl.ds(0, EW)], exp0_h.at[pl.ds(base, EW)])
        pltpu.sync_copy(ex1.at[pl.ds(0, EW)], exp1_h.at[pl.ds(base, EW)])
        pltpu.sync_copy(d0, pd0_h.at[pl.ds(wid * NP, NP)])
        pltpu.sync_copy(d1, pd1_h.at[pl.ds(wid * NP, NP)])
        pltpu.sync_copy(cn, pcn_h.at[pl.ds(wid * NP, NP)])

    return body(src, dst, aed, ptab)


def _weight_pass(src, dst, exp0, exp1, qtab):
    kfn = functools.partial(
        pl.kernel,
        out_type=(jax.ShapeDtypeStruct((2 * E,), jnp.float32),
                  jax.ShapeDtypeStruct((E,), jnp.float32),
                  jax.ShapeDtypeStruct((E,), jnp.int32)),
        mesh=_sc_mesh(),
        scratch_types=[
            pltpu.VMEM((NP,), jnp.float32),
            pltpu.VMEM((NP,), jnp.float32),
            pltpu.VMEM((NP,), jnp.float32),
            pltpu.VMEM((EWP,), jnp.int32),
            pltpu.VMEM((EWP,), jnp.int32),
            pltpu.VMEM((EWP,), jnp.float32),
            pltpu.VMEM((EWP,), jnp.float32),
            pltpu.VMEM((EWP,), jnp.float32),
            pltpu.VMEM((EWP,), jnp.float32),
            pltpu.VMEM((EWP,), jnp.float32),
            pltpu.VMEM((EWP,), jnp.int32),
        ],
        **_SC_PARAMS,
    )

    @kfn
    def body(src_h, dst_h, exp0_h, exp1_h, qtab_h, w01_h, wn_h, sd_h,
             q0, q1, disv, srcv, dstv, ex0, ex1, w0, w1, wn, sdv):
        _, _, wid, base = _worker_prologue()
        pltpu.sync_copy(qtab_h.at[pl.ds(0 * NP, NP)], q0)
        pltpu.sync_copy(qtab_h.at[pl.ds(1 * NP, NP)], q1)
        pltpu.sync_copy(qtab_h.at[pl.ds(2 * NP, NP)], disv)
        pltpu.sync_copy(src_h.at[pl.ds(base, EW)], srcv.at[pl.ds(0, EW)])
        pltpu.sync_copy(dst_h.at[pl.ds(base, EW)], dstv.at[pl.ds(0, EW)])
        pltpu.sync_copy(exp0_h.at[pl.ds(base, EW)], ex0.at[pl.ds(0, EW)])
        pltpu.sync_copy(exp1_h.at[pl.ds(base, EW)], ex1.at[pl.ds(0, EW)])

        iota = lax.iota(jnp.int32, 16)

        def ebody(b, _):
            off = b * 16
            m = (off + iota) < EW
            sv = jnp.where(m, srcv[pl.ds(off, 16)], 0)
            dv = jnp.where(m, dstv[pl.ds(off, 16)], 0)
            w0[pl.ds(off, 16)] = ex0[pl.ds(off, 16)] / plsc.load_gather(q0, [dv])
            w1[pl.ds(off, 16)] = ex1[pl.ds(off, 16)] / plsc.load_gather(q1, [dv])
            wn[pl.ds(off, 16)] = (plsc.load_gather(disv, [sv])
                                  * plsc.load_gather(disv, [dv]))
            sdv[pl.ds(off, 16)] = dv * 65536 + sv
            return 0

        lax.fori_loop(0, NB, ebody, 0)
        pltpu.sync_copy(w0.at[pl.ds(0, EW)], w01_h.at[pl.ds(base, EW)])
        pltpu.sync_copy(w1.at[pl.ds(0, EW)], w01_h.at[pl.ds(E + base, EW)])
        pltpu.sync_copy(wn.at[pl.ds(0, EW)], wn_h.at[pl.ds(base, EW)])
        pltpu.sync_copy(sdv.at[pl.ds(0, EW)], sd_h.at[pl.ds(base, EW)])

    return body(src, dst, exp0, exp1, qtab)


def _gat_aggregate(sd, w01, xwt):
    kfn = functools.partial(
        pl.kernel,
        out_type=jax.ShapeDtypeStruct((HC * NP,), jnp.float32),
        mesh=_sc_mesh(),
        scratch_types=(
            [pltpu.VMEM((NP,), jnp.float32) for _ in range(2 * WC)]
            + [pltpu.VMEM((EC,), jnp.int32), pltpu.VMEM((EC,), jnp.float32),
               pltpu.VMEM((EC,), jnp.int32), pltpu.VMEM((EC,), jnp.float32),
               pltpu.SemaphoreType.DMA, pltpu.SemaphoreType.DMA]),
        **_SC_PARAMS,
    )

    @kfn
    def body(sd_h, w01_h, xwt_h, agg_h,
             xc0, xc1, xc2, xc3, ac0, ac1, ac2, ac3,
             sb0, wb0, sb1, wb1, sem0, sem1):
        _, _, wid, _ = _worker_prologue()
        xc = [xc0, xc1, xc2, xc3]
        ac = [ac0, ac1, ac2, ac3]
        bufs = [(sb0, wb0, sem0), (sb1, wb1, sem1)]
        zf = jnp.zeros((16,), jnp.float32)
        woff = jnp.where(wid < 16, 0, E)

        def issue(par, eb):
            sb, wb, sem = bufs[par]
            pltpu.async_copy(sd_h.at[pl.ds(eb, EC)], sb, sem)
            pltpu.async_copy(w01_h.at[pl.ds(woff + eb, EC)], wb, sem)

        def drain(par):
            sb, wb, sem = bufs[par]
            pltpu.make_async_copy(sd_h.at[pl.ds(0, EC)], sb, sem).wait()
            pltpu.make_async_copy(w01_h.at[pl.ds(0, EC)], wb, sem).wait()

        def process(par):
            sb, wb, _ = bufs[par]

            @plsc.parallel_loop(0, EC, 16, unroll=8)
            def _(off):
                sd_v = sb[pl.ds(off, 16)]
                sv = jnp.bitwise_and(sd_v, 65535)
                dv = lax.shift_right_logical(sd_v, 16)
                wv = wb[pl.ds(off, 16)]
                for k in range(WC):
                    val = plsc.load_gather(xc[k], [sv]) * wv
                    plsc.addupdate_scatter(ac[k], [dv], val)

        for p in range(16 // WC):
            col0 = wid * 16 + p * WC
            for k in range(WC):
                pltpu.sync_copy(xwt_h.at[pl.ds((col0 + k) * NP, NP)], xc[k])

            @plsc.parallel_loop(0, NP, 16, unroll=4)
            def _(off):
                for k in range(WC):
                    ac[k][pl.ds(off, 16)] = zf

            issue(0, 0)

            def cbody(ci, _):
                nxt = ci + 1

                @pl.when(ci % 2 == 0)
                def _():
                    @pl.when(nxt < NCH)
                    def _():
                        issue(1, nxt * EC)
                    drain(0)
                    process(0)

                @pl.when(ci % 2 == 1)
                def _():
                    @pl.when(nxt < NCH)
                    def _():
                        issue(0, nxt * EC)
                    drain(1)
                    process(1)

                return 0

            lax.fori_loop(0, NCH, cbody, 0)
            for k in range(WC):
                pltpu.sync_copy(ac[k], agg_h.at[pl.ds((col0 + k) * NP, NP)])

    return body(sd, w01, xwt)


def _gcn_aggregate(sd, wn, qtab, hwt):
    kfn = functools.partial(
        pl.kernel,
        out_type=jax.ShapeDtypeStruct((C * NP,), jnp.float32),
        mesh=_sc_mesh(),
        scratch_types=(
            [pltpu.VMEM((NP,), jnp.float32) for _ in range(2 * WC)]
            + [pltpu.VMEM((NP,), jnp.float32),
               pltpu.VMEM((EC,), jnp.int32), pltpu.VMEM((EC,), jnp.float32),
               pltpu.VMEM((EC,), jnp.int32), pltpu.VMEM((EC,), jnp.float32),
               pltpu.SemaphoreType.DMA, pltpu.SemaphoreType.DMA]),
        **_SC_PARAMS,
    )

    @kfn
    def body(sd_h, wn_h, qtab_h, hwt_h, agg_h,
             xc0, xc1, xc2, xc3, ac0, ac1, ac2, ac3, swv,
             sb0, wb0, sb1, wb1, sem0, sem1):
        _, _, wid, _ = _worker_prologue()
        xc = [xc0, xc1, xc2, xc3]
        ac = [ac0, ac1, ac2, ac3]
        bufs = [(sb0, wb0, sem0), (sb1, wb1, sem1)]
        pltpu.sync_copy(qtab_h.at[pl.ds(3 * NP, NP)], swv)
        zf = jnp.zeros((16,), jnp.float32)

        def issue(par, eb):
            sb, wb, sem = bufs[par]
            pltpu.async_copy(sd_h.at[pl.ds(eb, EC)], sb, sem)
            pltpu.async_copy(wn_h.at[pl.ds(eb, EC)], wb, sem)

        def drain(par):
            sb, wb, sem = bufs[par]
            pltpu.make_async_copy(sd_h.at[pl.ds(0, EC)], sb, sem).wait()
            pltpu.make_async_copy(wn_h.at[pl.ds(0, EC)], wb, sem).wait()

        def process(par):
            sb, wb, _ = bufs[par]

            @plsc.parallel_loop(0, EC, 16, unroll=8)
            def _(off):
                sd_v = sb[pl.ds(off, 16)]
                sv = jnp.bitwise_and(sd_v, 65535)
                dv = lax.shift_right_logical(sd_v, 16)
                wv = wb[pl.ds(off, 16)]
                for k in range(WC):
                    val = plsc.load_gather(xc[k], [sv]) * wv
                    plsc.addupdate_scatter(ac[k], [dv], val)

        for p in range(8 // WC):
            col0 = wid * 8 + p * WC
            for k in range(WC):
                pltpu.sync_copy(hwt_h.at[pl.ds((col0 + k) * NP, NP)], xc[k])

            @plsc.parallel_loop(0, NP, 16, unroll=4)
            def _(off):
                for k in range(WC):
                    ac[k][pl.ds(off, 16)] = zf

            issue(0, 0)

            def cbody(ci, _):
                nxt = ci + 1

                @pl.when(ci % 2 == 0)
                def _():
                    @pl.when(nxt < NCH)
                    def _():
                        issue(1, nxt * EC)
                    drain(0)
                    process(0)

                @pl.when(ci % 2 == 1)
                def _():
                    @pl.when(nxt < NCH)
                    def _():
                        issue(0, nxt * EC)
                    drain(1)
                    process(1)

                return 0

            lax.fori_loop(0, NCH, cbody, 0)

            @plsc.parallel_loop(0, NP, 16, unroll=4)
            def _(off):
                swl = swv[pl.ds(off, 16)]
                for k in range(WC):
                    ac[k][pl.ds(off, 16)] = (ac[k][pl.ds(off, 16)]
                                             + swl * xc[k][pl.ds(off, 16)])

            for k in range(WC):
                pltpu.sync_copy(ac[k], agg_h.at[pl.ds((col0 + k) * NP, NP)])

    return body(sd, wn, qtab, hwt)


def kernel(x, edge_index, edge_attr, W_gat, b_gat, att_src, att_dst, att_edge,
           W_edge, gamma1, beta1, gamma2, beta2, W1, b1, W2, b2):
    src = edge_index[0]
    dst = edge_index[1]
    xwt, ptab = _xw_and_tables(W_gat.T, x, W_gat, att_src, att_dst)
    xwt = xwt.reshape(HC * NP)
    ptab = ptab.reshape(8 * NP)
    aed = _edge_tables(edge_attr, W_edge, att_edge).reshape(8 * E)
    exp0, exp1, pd0, pd1, pcnt = _exp_denom_pass(src, dst, aed, ptab)
    qtab = _reduce_tables(pd0.reshape(NW, NP), pd1.reshape(NW, NP),
                          pcnt.reshape(NW, NP)).reshape(8 * NP)
    w01, wn, sd = _weight_pass(src, dst, exp0, exp1, qtab)
    aggt = _gat_aggregate(sd, w01, xwt).reshape(HC, NP)
    h1wt = _fuse_matmul_t(aggt, b_gat[:, None], gamma1[:, None],
                          beta1[:, None], W1.T).reshape(C * NP)
    agg1t = _gcn_aggregate(sd, wn, qtab, h1wt).reshape(C, NP)
    h2wt = _fuse_matmul_t(agg1t, b1[:, None], gamma2[:, None],
                          beta2[:, None], W2.T).reshape(C * NP)
    agg2t = _gcn_aggregate(sd, wn, qtab, h2wt).reshape(C, NP)
    node_emb, graph_emb = _final(agg2t, x, b2[:, None])
    return (node_emb, graph_emb)


# speedup vs baseline: 1.0209x; 1.0047x over previous
import functools

import jax
import jax.numpy as jnp
from jax import lax
from jax.experimental import pallas as pl
from jax.experimental.pallas import tpu as pltpu
from jax.experimental.pallas import tpu_sc as plsc

N = 10000
E = 160000
F_IN = 256
F_EDGE = 16
H = 2
C = 256
HC = H * C
EPS = 1e-5
INV_BN = float(1.0 / (1.0 + EPS) ** 0.5)

NP = 10240
NW = 32
EW = E // NW
EWP = EW + 16
NB = (EW + 15) // 16

EC = 4000
NCH = E // EC
WC = 4

_SC_PARAMS = dict(
    compiler_params=pltpu.CompilerParams(needs_layout_passes=False),
)


def _k1a_body(wt_ref, x_ref, w_ref, asrc_ref, adst_ref, o_ref, p_ref):
    xb = x_ref[...]
    o_ref[...] = lax.dot_general(
        wt_ref[...], xb, (((1,), (1,)), ((), ())),
        preferred_element_type=jnp.float32)
    w = w_ref[...]
    us0 = jnp.dot(w[:, :C], asrc_ref[0, :], preferred_element_type=jnp.float32)
    us1 = jnp.dot(w[:, C:], asrc_ref[1, :], preferred_element_type=jnp.float32)
    ud0 = jnp.dot(w[:, :C], adst_ref[0, :], preferred_element_type=jnp.float32)
    ud1 = jnp.dot(w[:, C:], adst_ref[1, :], preferred_element_type=jnp.float32)
    p_ref[0:1, :] = jnp.dot(xb, us0, preferred_element_type=jnp.float32)[None, :]
    p_ref[1:2, :] = jnp.dot(xb, us1, preferred_element_type=jnp.float32)[None, :]
    p_ref[2:3, :] = jnp.dot(xb, ud0, preferred_element_type=jnp.float32)[None, :]
    p_ref[3:4, :] = jnp.dot(xb, ud1, preferred_element_type=jnp.float32)[None, :]
    p_ref[4:8, :] = jnp.zeros((4, p_ref.shape[1]), jnp.float32)


def _xw_and_tables(wt, x, W_gat, att_src, att_dst):
    bn = NP // 8
    f = wt.shape[0]
    return pl.pallas_call(
        _k1a_body,
        grid=(8,),
        in_specs=[pl.BlockSpec((f, F_IN), lambda i: (0, 0)),
                  pl.BlockSpec((bn, F_IN), lambda i: (i, 0)),
                  pl.BlockSpec((F_IN, HC), lambda i: (0, 0)),
                  pl.BlockSpec((H, C), lambda i: (0, 0)),
                  pl.BlockSpec((H, C), lambda i: (0, 0))],
        out_specs=[pl.BlockSpec((f, bn), lambda i: (0, i)),
                   pl.BlockSpec((8, bn), lambda i: (0, i))],
        out_shape=[jax.ShapeDtypeStruct((f, NP), jnp.float32),
                   jax.ShapeDtypeStruct((8, NP), jnp.float32)],
    )(wt, x, W_gat, att_src, att_dst)


def _k1c_body(ea_ref, w_ref, ae_ref, o_ref):
    w = w_ref[...]
    ve0 = jnp.dot(w[:, :C], ae_ref[0, :], preferred_element_type=jnp.float32)
    ve1 = jnp.dot(w[:, C:], ae_ref[1, :], preferred_element_type=jnp.float32)
    ea = ea_ref[...]
    o_ref[0:1, :] = jnp.dot(ea, ve0, preferred_element_type=jnp.float32)[None, :]
    o_ref[1:2, :] = jnp.dot(ea, ve1, preferred_element_type=jnp.float32)[None, :]
    o_ref[2:8, :] = jnp.zeros((6, o_ref.shape[1]), jnp.float32)


def _edge_tables(edge_attr, W_edge, att_edge):
    be = 1280
    return pl.pallas_call(
        _k1c_body,
        grid=(E // be,),
        in_specs=[pl.BlockSpec((be, F_EDGE), lambda i: (i, 0)),
                  pl.BlockSpec((F_EDGE, HC), lambda i: (0, 0)),
                  pl.BlockSpec((H, C), lambda i: (0, 0))],
        out_specs=pl.BlockSpec((8, be), lambda i: (0, i)),
        out_shape=jax.ShapeDtypeStruct((8, E), jnp.float32),
    )(edge_attr, W_edge, att_edge)


def _k2b_body(pd0_ref, pd1_ref, pc_ref, o_ref):
    s0 = jnp.sum(pd0_ref[...], axis=0)
    s1 = jnp.sum(pd1_ref[...], axis=0)
    deg = jnp.sum(pc_ref[...], axis=0) + 1.0
    o_ref[0:1, :] = (s0 + 1e-16)[None, :]
    o_ref[1:2, :] = (s1 + 1e-16)[None, :]
    o_ref[2:3, :] = lax.rsqrt(deg)[None, :]
    o_ref[3:4, :] = (1.0 / deg)[None, :]
    o_ref[4:8, :] = jnp.zeros((4, o_ref.shape[1]), jnp.float32)


def _reduce_tables(pd0, pd1, pcnt):
    bn = NP // 8
    return pl.pallas_call(
        _k2b_body,
        grid=(8,),
        in_specs=[pl.BlockSpec((NW, bn), lambda i: (0, i)),
                  pl.BlockSpec((NW, bn), lambda i: (0, i)),
                  pl.BlockSpec((NW, bn), lambda i: (0, i))],
        out_specs=pl.BlockSpec((8, bn), lambda i: (0, i)),
        out_shape=jax.ShapeDtypeStruct((8, NP), jnp.float32),
    )(pd0, pd1, pcnt)


def _k4_body(agg_ref, b_ref, g_ref, be_ref, wt_ref, o_ref):
    hb = agg_ref[...] + b_ref[...]
    hb = hb * (INV_BN * g_ref[...]) + be_ref[...]
    hb = jnp.where(hb > 0, hb, jnp.exp(hb) - 1.0)
    o_ref[...] = lax.dot_general(
        wt_ref[...], hb, (((1,), (0,)), ((), ())),
        preferred_element_type=jnp.float32)


def _fuse_matmul_t(aggt, bcol, gcol, becol, wt):
    bn = NP // 8
    f = aggt.shape[0]
    fo = wt.shape[0]
    return pl.pallas_call(
        _k4_body,
        grid=(8,),
        in_specs=[pl.BlockSpec((f, bn), lambda i: (0, i)),
                  pl.BlockSpec((f, 1), lambda i: (0, 0)),
                  pl.BlockSpec((f, 1), lambda i: (0, 0)),
                  pl.BlockSpec((f, 1), lambda i: (0, 0)),
                  pl.BlockSpec((fo, f), lambda i: (0, 0))],
        out_specs=pl.BlockSpec((fo, bn), lambda i: (0, i)),
        out_shape=jax.ShapeDtypeStruct((fo, NP), jnp.float32),
    )(aggt, bcol, gcol, becol, wt)


def _k8_body(agg_ref, x_ref, b_ref, o_ref, m_ref):
    net = agg_ref[...] + b_ref[...]
    r = lax.broadcasted_iota(jnp.int32, (C, C), 0)
    c = lax.broadcasted_iota(jnp.int32, (C, C), 1)
    eye = (r == c).astype(jnp.float32)
    ne = lax.dot_general(net, eye, (((0,), (0,)), ((), ())),
                         preferred_element_type=jnp.float32)
    ne = ne + x_ref[...]
    o_ref[...] = ne
    i = pl.program_id(0)
    bn = ne.shape[0]

    @pl.when(i == 0)
    def _():
        m_ref[...] = jnp.zeros((1, C), jnp.float32)

    rows = lax.broadcasted_iota(jnp.int32, (bn, C), 0) + i * bn
    m_ref[...] += jnp.sum(jnp.where(rows < N, ne, 0.0), axis=0,
                          keepdims=True)

    @pl.when(i == pl.num_programs(0) - 1)
    def _():
        m_ref[...] *= (1.0 / N)


def _final(agg2t, x, b2col):
    bn = NP // 8
    return pl.pallas_call(
        _k8_body,
        grid=(8,),
        in_specs=[pl.BlockSpec((C, bn), lambda i: (0, i)),
                  pl.BlockSpec((bn, C), lambda i: (i, 0)),
                  pl.BlockSpec((C, 1), lambda i: (0, 0))],
        out_specs=[pl.BlockSpec((bn, C), lambda i: (i, 0)),
                   pl.BlockSpec((1, C), lambda i: (0, 0))],
        out_shape=[jax.ShapeDtypeStruct((N, C), jnp.float32),
                   jax.ShapeDtypeStruct((1, C), jnp.float32)],
    )(agg2t, x, b2col)


def _sc_mesh():
    return plsc.VectorSubcoreMesh(core_axis_name="c", subcore_axis_name="s")


def _worker_prologue():
    cid = lax.axis_index("c")
    sid = lax.axis_index("s")
    wid = sid * 2 + cid
    return cid, sid, wid, wid * EW


def _exp_denom_pass(src, dst, aed, ptab):
    kfn = functools.partial(
        pl.kernel,
        out_type=(jax.ShapeDtypeStruct((E,), jnp.float32),
                  jax.ShapeDtypeStruct((E,), jnp.float32),
                  jax.ShapeDtypeStruct((NW * NP,), jnp.float32),
                  jax.ShapeDtypeStruct((NW * NP,), jnp.float32),
                  jax.ShapeDtypeStruct((NW * NP,), jnp.float32)),
        mesh=_sc_mesh(),
        scratch_types=[
            pltpu.VMEM((NP,), jnp.float32),
            pltpu.VMEM((NP,), jnp.float32),
            pltpu.VMEM((NP,), jnp.float32),
            pltpu.VMEM((NP,), jnp.float32),
            pltpu.VMEM((EWP,), jnp.int32),
            pltpu.VMEM((EWP,), jnp.int32),
            pltpu.VMEM((EWP,), jnp.float32),
            pltpu.VMEM((EWP,), jnp.float32),
            pltpu.VMEM((EWP,), jnp.float32),
            pltpu.VMEM((EWP,), jnp.float32),
            pltpu.VMEM((NP,), jnp.float32),
            pltpu.VMEM((NP,), jnp.float32),
            pltpu.VMEM((NP,), jnp.float32),
        ],
        **_SC_PARAMS,
    )

    @kfn
    def body(src_h, dst_h, aed_h, ptab_h, exp0_h, exp1_h, pd0_h, pd1_h, pcn_h,
             as0, as1, ad0, ad1, srcv, dstv, ae0, ae1, ex0, ex1, d0, d1, cn):
        _, _, wid, base = _worker_prologue()
        pltpu.sync_copy(ptab_h.at[pl.ds(0 * NP, NP)], as0)
        pltpu.sync_copy(ptab_h.at[pl.ds(1 * NP, NP)], as1)
        pltpu.sync_copy(ptab_h.at[pl.ds(2 * NP, NP)], ad0)
        pltpu.sync_copy(ptab_h.at[pl.ds(3 * NP, NP)], ad1)
        pltpu.sync_copy(src_h.at[pl.ds(base, EW)], srcv.at[pl.ds(0, EW)])
        pltpu.sync_copy(dst_h.at[pl.ds(base, EW)], dstv.at[pl.ds(0, EW)])
        pltpu.sync_copy(aed_h.at[pl.ds(0 * E + base, EW)], ae0.at[pl.ds(0, EW)])
        pltpu.sync_copy(aed_h.at[pl.ds(1 * E + base, EW)], ae1.at[pl.ds(0, EW)])

        zf = jnp.zeros((16,), jnp.float32)

        @plsc.parallel_loop(0, NP, 16, unroll=4)
        def _(off):
            d0[pl.ds(off, 16)] = zf
            d1[pl.ds(off, 16)] = zf
            cn[pl.ds(off, 16)] = zf

        iota = lax.iota(jnp.int32, 16)
        ones = jnp.ones((16,), jnp.float32)

        def ebody(b, _):
            off = b * 16
            m = (off + iota) < EW
            sv = jnp.where(m, srcv[pl.ds(off, 16)], 0)
            dv = jnp.where(m, dstv[pl.ds(off, 16)], 0)
            a0 = (plsc.load_gather(as0, [sv]) + plsc.load_gather(ad0, [dv])
                  + ae0[pl.ds(off, 16)])
            a1 = (plsc.load_gather(as1, [sv]) + plsc.load_gather(ad1, [dv])
                  + ae1[pl.ds(off, 16)])
            a0 = jnp.where(a0 >= 0.0, a0, a0 * 0.2)
            a1 = jnp.where(a1 >= 0.0, a1, a1 * 0.2)
            e0 = jnp.exp(a0)
            e1 = jnp.exp(a1)
            ex0[pl.ds(off, 16)] = e0
            ex1[pl.ds(off, 16)] = e1
            plsc.addupdate_scatter(d0, [dv], e0, mask=m)
            plsc.addupdate_scatter(d1, [dv], e1, mask=m)
            plsc.addupdate_scatter(cn, [dv], ones, mask=m)
            return 0

        lax.fori_loop(0, NB, ebody, 0)
        pltpu.sync_copy(ex0.at[pl.ds(0, EW)], exp0_h.at[pl.ds(base, EW)])
        pltpu.sync_copy(ex1.at[pl.ds(0, EW)], exp1_h.at[pl.ds(base, EW)])
        pltpu.sync_copy(d0, pd0_h.at[pl.ds(wid * NP, NP)])
        pltpu.sync_copy(d1, pd1_h.at[pl.ds(wid * NP, NP)])
        pltpu.sync_copy(cn, pcn_h.at[pl.ds(wid * NP, NP)])

    return body(src, dst, aed, ptab)


def _weight_pass(src, dst, exp0, exp1, qtab):
    kfn = functools.partial(
        pl.kernel,
        out_type=(jax.ShapeDtypeStruct((2 * E,), jnp.float32),
                  jax.ShapeDtypeStruct((E,), jnp.float32),
                  jax.ShapeDtypeStruct((E,), jnp.int32)),
        mesh=_sc_mesh(),
        scratch_types=[
            pltpu.VMEM((NP,), jnp.float32),
            pltpu.VMEM((NP,), jnp.float32),
            pltpu.VMEM((NP,), jnp.float32),
            pltpu.VMEM((EWP,), jnp.int32),
            pltpu.VMEM((EWP,), jnp.int32),
            pltpu.VMEM((EWP,), jnp.float32),
            pltpu.VMEM((EWP,), jnp.float32),
            pltpu.VMEM((EWP,), jnp.float32),
            pltpu.VMEM((EWP,), jnp.float32),
            pltpu.VMEM((EWP,), jnp.float32),
            pltpu.VMEM((EWP,), jnp.int32),
        ],
        **_SC_PARAMS,
    )

    @kfn
    def body(src_h, dst_h, exp0_h, exp1_h, qtab_h, w01_h, wn_h, sd_h,
             q0, q1, disv, srcv, dstv, ex0, ex1, w0, w1, wn, sdv):
        _, _, wid, base = _worker_prologue()
        pltpu.sync_copy(qtab_h.at[pl.ds(0 * NP, NP)], q0)
        pltpu.sync_copy(qtab_h.at[pl.ds(1 * NP, NP)], q1)
        pltpu.sync_copy(qtab_h.at[pl.ds(2 * NP, NP)], disv)
        pltpu.sync_copy(src_h.at[pl.ds(base, EW)], srcv.at[pl.ds(0, EW)])
        pltpu.sync_copy(dst_h.at[pl.ds(base, EW)], dstv.at[pl.ds(0, EW)])
        pltpu.sync_copy(exp0_h.at[pl.ds(base, EW)], ex0.at[pl.ds(0, EW)])
        pltpu.sync_copy(exp1_h.at[pl.ds(base, EW)], ex1.at[pl.ds(0, EW)])

        iota = lax.iota(jnp.int32, 16)

        def ebody(b, _):
            off = b * 16
            m = (off + iota) < EW
            sv = jnp.where(m, srcv[pl.ds(off, 16)], 0)
            dv = jnp.where(m, dstv[pl.ds(off, 16)], 0)
            w0[pl.ds(off, 16)] = ex0[pl.ds(off, 16)] / plsc.load_gather(q0, [dv])
            w1[pl.ds(off, 16)] = ex1[pl.ds(off, 16)] / plsc.load_gather(q1, [dv])
            wn[pl.ds(off, 16)] = (plsc.load_gather(disv, [sv])
                                  * plsc.load_gather(disv, [dv]))
            sdv[pl.ds(off, 16)] = dv * 65536 + sv
            return 0

        lax.fori_loop(0, NB, ebody, 0)
        pltpu.sync_copy(w0.at[pl.ds(0, EW)], w01_h.at[pl.ds(base, EW)])
        pltpu.sync_copy(w1.at[pl.ds(0, EW)], w01_h.at[pl.ds(E + base, EW)])
        pltpu.sync_copy(wn.at[pl.ds(0, EW)], wn_h.at[pl.ds(base, EW)])
        pltpu.sync_copy(sdv.at[pl.ds(0, EW)], sd_h.at[pl.ds(base, EW)])

    return body(src, dst, exp0, exp1, qtab)


def _gat_aggregate(sd, w01, xwt):
    kfn = functools.partial(
        pl.kernel,
        out_type=jax.ShapeDtypeStruct((HC * NP,), jnp.float32),
        mesh=_sc_mesh(),
        scratch_types=(
            [pltpu.VMEM((NP,), jnp.float32) for _ in range(2 * WC)]
            + [pltpu.VMEM((EC,), jnp.int32), pltpu.VMEM((EC,), jnp.float32),
               pltpu.VMEM((EC,), jnp.int32), pltpu.VMEM((EC,), jnp.float32),
               pltpu.SemaphoreType.DMA, pltpu.SemaphoreType.DMA]),
        **_SC_PARAMS,
    )

    @kfn
    def body(sd_h, w01_h, xwt_h, agg_h,
             xc0, xc1, xc2, xc3, ac0, ac1, ac2, ac3,
             sb0, wb0, sb1, wb1, sem0, sem1):
        _, _, wid, _ = _worker_prologue()
        xc = [xc0, xc1, xc2, xc3]
        ac = [ac0, ac1, ac2, ac3]
        bufs = [(sb0, wb0, sem0), (sb1, wb1, sem1)]
        zf = jnp.zeros((16,), jnp.float32)
        woff = jnp.where(wid < 16, 0, E)

        def issue(par, eb):
            sb, wb, sem = bufs[par]
            pltpu.async_copy(sd_h.at[pl.ds(eb, EC)], sb, sem)
            pltpu.async_copy(w01_h.at[pl.ds(woff + eb, EC)], wb, sem)

        def drain(par):
            sb, wb, sem = bufs[par]
            pltpu.make_async_copy(sd_h.at[pl.ds(0, EC)], sb, sem).wait()
            pltpu.make_async_copy(w01_h.at[pl.ds(0, EC)], wb, sem).wait()

        def process(par):
            sb, wb, _ = bufs[par]

            @plsc.parallel_loop(0, EC, 16, unroll=8)
            def _(off):
                sd_v = sb[pl.ds(off, 16)]
                sv = jnp.bitwise_and(sd_v, 65535)
                dv = lax.shift_right_logical(sd_v, 16)
                wv = wb[pl.ds(off, 16)]
                for k in range(WC):
                    val = plsc.load_gather(xc[k], [sv]) * wv
                    plsc.addupdate_scatter(ac[k], [dv], val)

        for p in range(16 // WC):
            col0 = wid * 16 + p * WC
            for k in range(WC):
                pltpu.sync_copy(xwt_h.at[pl.ds((col0 + k) * NP, NP)], xc[k])

            @plsc.parallel_loop(0, NP, 16, unroll=4)
            def _(off):
                for k in range(WC):
                    ac[k][pl.ds(off, 16)] = zf

            issue(0, 0)

            def cbody(ci, _):
                nxt = ci + 1

                @pl.when(ci % 2 == 0)
                def _():
                    @pl.when(nxt < NCH)
                    def _():
                        issue(1, nxt * EC)
                    drain(0)
                    process(0)

                @pl.when(ci % 2 == 1)
                def _():
                    @pl.when(nxt < NCH)
                    def _():
                        issue(0, nxt * EC)
                    drain(1)
                    process(1)

                return 0

            lax.fori_loop(0, NCH, cbody, 0)
            for k in range(WC):
                pltpu.sync_copy(ac[k], agg_h.at[pl.ds((col0 + k) * NP, NP)])

    return body(sd, w01, xwt)


def _gcn_aggregate(sd, wn, qtab, hwt):
    kfn = functools.partial(
        pl.kernel,
        out_type=jax.ShapeDtypeStruct((C * NP,), jnp.float32),
        mesh=_sc_mesh(),
        scratch_types=(
            [pltpu.VMEM((NP,), jnp.float32) for _ in range(2 * WC)]
            + [pltpu.VMEM((NP,), jnp.float32),
               pltpu.VMEM((EC,), jnp.int32), pltpu.VMEM((EC,), jnp.float32),
               pltpu.VMEM((EC,), jnp.int32), pltpu.VMEM((EC,), jnp.float32),
               pltpu.SemaphoreType.DMA, pltpu.SemaphoreType.DMA]),
        **_SC_PARAMS,
    )

    @kfn
    def body(sd_h, wn_h, qtab_h, hwt_h, agg_h,
             xc0, xc1, xc2, xc3, ac0, ac1, ac2, ac3, swv,
             sb0, wb0, sb1, wb1, sem0, sem1):
        _, _, wid, _ = _worker_prologue()
        xc = [xc0, xc1, xc2, xc3]
        ac = [ac0, ac1, ac2, ac3]
        bufs = [(sb0, wb0, sem0), (sb1, wb1, sem1)]
        pltpu.sync_copy(qtab_h.at[pl.ds(3 * NP, NP)], swv)
        zf = jnp.zeros((16,), jnp.float32)

        def issue(par, eb):
            sb, wb, sem = bufs[par]
            pltpu.async_copy(sd_h.at[pl.ds(eb, EC)], sb, sem)
            pltpu.async_copy(wn_h.at[pl.ds(eb, EC)], wb, sem)

        def drain(par):
            sb, wb, sem = bufs[par]
            pltpu.make_async_copy(sd_h.at[pl.ds(0, EC)], sb, sem).wait()
            pltpu.make_async_copy(wn_h.at[pl.ds(0, EC)], wb, sem).wait()

        def process(par):
            sb, wb, _ = bufs[par]

            @plsc.parallel_loop(0, EC, 16, unroll=8)
            def _(off):
                sd_v = sb[pl.ds(off, 16)]
                sv = jnp.bitwise_and(sd_v, 65535)
                dv = lax.shift_right_logical(sd_v, 16)
                wv = wb[pl.ds(off, 16)]
                for k in range(WC):
                    val = plsc.load_gather(xc[k], [sv]) * wv
                    plsc.addupdate_scatter(ac[k], [dv], val)

        for p in range(8 // WC):
            col0 = wid * 8 + p * WC
            for k in range(WC):
                pltpu.sync_copy(hwt_h.at[pl.ds((col0 + k) * NP, NP)], xc[k])

            @plsc.parallel_loop(0, NP, 16, unroll=4)
            def _(off):
                for k in range(WC):
                    ac[k][pl.ds(off, 16)] = zf

            issue(0, 0)

            def cbody(ci, _):
                nxt = ci + 1

                @pl.when(ci % 2 == 0)
                def _():
                    @pl.when(nxt < NCH)
                    def _():
                        issue(1, nxt * EC)
                    drain(0)
                    process(0)

                @pl.when(ci % 2 == 1)
                def _():
                    @pl.when(nxt < NCH)
                    def _():
                        issue(0, nxt * EC)
                    drain(1)
                    process(1)

                return 0

            lax.fori_loop(0, NCH, cbody, 0)

            @plsc.parallel_loop(0, NP, 16, unroll=4)
            def _(off):
                swl = swv[pl.ds(off, 16)]
                for k in range(WC):
                    ac[k][pl.ds(off, 16)] = (ac[k][pl.ds(off, 16)]
                                             + swl * xc[k][pl.ds(off, 16)])

            for k in range(WC):
                pltpu.sync_copy(ac[k], agg_h.at[pl.ds((col0 + k) * NP, NP)])

    return body(sd, wn, qtab, hwt)


def kernel(x, edge_index, edge_attr, W_gat, b_gat, att_src, att_dst, att_edge,
           W_edge, gamma1, beta1, gamma2, beta2, W1, b1, W2, b2):
    src = edge_index[0]
    dst = edge_index[1]
    xwt, ptab = _xw_and_tables(W_gat.T, x, W_gat, att_src, att_dst)
    xwt = xwt.reshape(HC * NP)
    ptab = ptab.reshape(8 * NP)
    aed = _edge_tables(edge_attr, W_edge, att_edge).reshape(8 * E)
    exp0, exp1, pd0, pd1, pcnt = _exp_denom_pass(src, dst, aed, ptab)
    qtab = _reduce_tables(pd0.reshape(NW, NP), pd1.reshape(NW, NP),
                          pcnt.reshape(NW, NP)).reshape(8 * NP)
    w01, wn, sd = _weight_pass(src, dst, exp0, exp1, qtab)
    aggt = _gat_aggregate(sd, w01, xwt).reshape(HC, NP)
    h1wt = _fuse_matmul_t(aggt, b_gat[:, None], gamma1[:, None],
                          beta1[:, None], W1.T).reshape(C * NP)
    agg1t = _gcn_aggregate(sd, wn, qtab, h1wt).reshape(C, NP)
    h2wt = _fuse_matmul_t(agg1t, b1[:, None], gamma2[:, None],
                          beta2[:, None], W2.T).reshape(C * NP)
    agg2t = _gcn_aggregate(sd, wn, qtab, h2wt).reshape(C, NP)
    node_emb, graph_emb = _final(agg2t, x, b2[:, None])
    return (node_emb, graph_emb)
